# baseline (speedup 1.0000x reference)
import functools

import jax
import jax.numpy as jnp
from jax import lax
from jax.experimental import pallas as pl
from jax.experimental.pallas import tpu as pltpu
from jax.experimental.pallas import tpu_sc as plsc

_N, _G, _E, _D = 10000, 32, 320000, 128
_NP = 10240
_R = 2000
_B = 80
_EPT = _E // 32
_RPS = _NP // 16
_EB = _E // _B
_CH = 2000
_CR = _CH // _B
_NCH = _EPT // _CH


def _ssilu(x):
    return (x * jax.nn.sigmoid(x)) * (1.0 / 0.6)


def _node_body(an_ref, n2g_ref, pos_ref, emb_ref, we_ref, be_ref, cell_ref,
               es_ref, ed_ref,
               h_ref, m_ref, w1_ref, w2_ref, pk_ref):
    c9 = cell_ref[...]
    a, b, c, d, e, f, g, h, i = [c9[:, k:k + 1] for k in range(9)]
    c11 = e * i - f * h
    c12 = -(d * i - f * g)
    c13 = d * h - e * g
    det = a * c11 + b * c12 + c * c13
    inv9 = jnp.concatenate([
        c11, -(b * i - c * h), b * f - c * e,
        c12, a * i - c * g, -(a * f - c * d),
        c13, -(a * h - b * g), a * e - b * d,
    ], axis=1) / det

    an = an_ref[...]
    oh = (an == lax.broadcasted_iota(jnp.int32, (_R, 100), 1)).astype(jnp.float32)
    hh = jnp.dot(oh, emb_ref[...], preferred_element_type=jnp.float32)
    h_ref[...] = hh
    m_ref[...] = _ssilu(jnp.dot(hh, we_ref[...], preferred_element_type=jnp.float32)
                        + be_ref[...])

    n2g = n2g_ref[...]
    ohg = (n2g == lax.broadcasted_iota(jnp.int32, (_R, _G), 1)).astype(jnp.float32)
    ci = jnp.dot(ohg, inv9, preferred_element_type=jnp.float32)
    pos = pos_ref[...]
    q = []
    for j in range(3):
        v = (pos[:, 0:1] * ci[:, j:j + 1]
             + pos[:, 1:2] * ci[:, 3 + j:4 + j]
             + pos[:, 2:3] * ci[:, 6 + j:7 + j])
        v = v - jnp.floor(v)
        q.append(jnp.minimum(jnp.floor(v * 65536.0), 65535.0).astype(jnp.int32))
    w1_ref[...] = q[0] + q[1] * 65536
    w2_ref[...] = q[2] + n2g * 65536

    pk_ref[...] = ed_ref[...] * 16384 + es_ref[...]


def _node_call(an, n2g, pos, emb, we, be, cell9, e2):
    grid = _N // _R
    return pl.pallas_call(
        _node_body,
        grid=(grid,),
        in_specs=[
            pl.BlockSpec((_R, 1), lambda i: (i, 0)),
            pl.BlockSpec((_R, 1), lambda i: (i, 0)),
            pl.BlockSpec((_R, 3), lambda i: (i, 0)),
            pl.BlockSpec((100, _D), lambda i: (0, 0)),
            pl.BlockSpec((_D, _D), lambda i: (0, 0)),
            pl.BlockSpec((1, _D), lambda i: (0, 0)),
            pl.BlockSpec((_G, 9), lambda i: (0, 0)),
            pl.BlockSpec((_EB // 5, _B), lambda i: (i, 0)),
            pl.BlockSpec((_EB // 5, _B), lambda i: (i + 5, 0)),
        ],
        out_specs=[
            pl.BlockSpec((_R, _D), lambda i: (i, 0)),
            pl.BlockSpec((_R, _D), lambda i: (i, 0)),
            pl.BlockSpec((_R, 1), lambda i: (i, 0)),
            pl.BlockSpec((_R, 1), lambda i: (i, 0)),
            pl.BlockSpec((_EB // 5, _B), lambda i: (i, 0)),
        ],
        out_shape=[
            jax.ShapeDtypeStruct((_N, _D), jnp.float32),
            jax.ShapeDtypeStruct((_N, _D), jnp.float32),
            jax.ShapeDtypeStruct((_N, 1), jnp.int32),
            jax.ShapeDtypeStruct((_N, 1), jnp.int32),
            jax.ShapeDtypeStruct((_EB, _B), jnp.int32),
        ],
    )(an, n2g, pos, emb, we, be, cell9, e2, e2)


def _splat_i32(v):
    return jnp.full((16,), v, dtype=jnp.int32)


def _edge_body(pk_hbm, w1_hbm, w2_hbm, cell_hbm, m_hbm, out_hbm,
               w1_v, w2_v, cell_v, pkbuf, cpk, cgate, sidx0, sidx1,
               didx0, didx1, rows0, rows1, agg_sh,
               gsem0, gsem1, ssem0, ssem1):
    cid = lax.axis_index("c")
    sid = lax.axis_index("s")
    wid = cid * 16 + sid

    pltpu.sync_copy(w1_hbm, w1_v)
    pltpu.sync_copy(w2_hbm, w2_v)
    pltpu.sync_copy(cell_hbm, cell_v)

    def _zf(r, carry):
        for q in range(_D // 16):
            rows0[r, pl.ds(q * 16, 16)] = jnp.zeros((16,), jnp.float32)
        return carry
    lax.fori_loop(0, _B, _zf, 0)
    row0 = sid * _RPS

    def _zc(t, carry):
        pltpu.sync_copy(rows0, agg_sh.at[pl.ds(row0 + t * _B, _B)])
        return carry
    lax.fori_loop(0, _RPS // _B, _zc, 0)
    plsc.subcore_barrier()

    sidx = (sidx0, sidx1)
    didx = (didx0, didx1)
    rows = (rows0, rows1)
    gsem = (gsem0, gsem1)
    ssem = (ssem0, ssem1)
    zi16 = jnp.zeros((16,), jnp.int32)
    zf16 = jnp.zeros((16,), jnp.float32)
    lane = lax.iota(jnp.int32, 16)
    u16scale = jnp.full((16,), 1.0 / 65536.0, jnp.float32)

    def build_idx(b, p):
        off = pl.multiple_of(b * _B, 16)
        def _bi(k, carry):
            v = cpk[pl.ds(off + k * 16, 16)]
            sidx[p][pl.ds(k * 16, 16)] = v & 16383
            didx[p][pl.ds(k * 16, 16)] = lax.shift_right_logical(v, 14)
            return carry
        lax.fori_loop(0, _B // 16, _bi, 0)

    def scale(b, p):
        boff = b * _B
        def _s(e4, carry):
            for u in range(4):
                ei = e4 * 4 + u
                gs = plsc.load_gather(cgate, [_splat_i32(boff + ei)])
                for q in range(_D // 16):
                    rows[p][ei, pl.ds(q * 16, 16)] = rows[p][ei, pl.ds(q * 16, 16)] * gs
            return carry
        lax.fori_loop(0, _B // 4, _s, 0)

    def _chunk(c, carry):
        pltpu.sync_copy(pk_hbm.at[pl.ds(wid * (_EPT // _B) + c * _CR, _CR)], pkbuf)

        def _row(r, wv):
            for k in range(_B // 16):
                v = pkbuf[r, pl.ds(k * 16, 16)]
                s16 = v & 16383
                d16 = lax.shift_right_logical(v, 14)
                ws1 = plsc.load_gather(w1_v, [s16])
                wd1 = plsc.load_gather(w1_v, [d16])
                ws2 = plsc.load_gather(w2_v, [s16])
                wd2 = plsc.load_gather(w2_v, [d16])
                df = []
                for (ws, wd, hi) in ((ws1, wd1, 0), (ws1, wd1, 1), (ws2, wd2, 0)):
                    if hi == 0:
                        qs = (ws & 65535).astype(jnp.float32)
                        qd = (wd & 65535).astype(jnp.float32)
                    else:
                        qs = lax.shift_right_logical(ws, 16).astype(jnp.float32)
                        qd = lax.shift_right_logical(wd, 16).astype(jnp.float32)
                    x = (qd - qs) * u16scale
                    x = jnp.where(x > 0.5, x - 1.0,
                                  jnp.where(x < -0.5, x + 1.0, x))
                    df.append(x)
                g9 = lax.shift_right_logical(ws2, 16) * 9
                s2 = jnp.full((16,), 1e-12, jnp.float32)
                for j3 in range(3):
                    dv = jnp.zeros((16,), jnp.float32)
                    for comp in range(3):
                        ce = plsc.load_gather(cell_v, [g9 + _splat_i32(comp * 3 + j3)])
                        dv = dv + df[comp] * ce
                    s2 = s2 + dv * dv
                keep = s2 < 16.0
                g = jnp.exp(s2 * (-1.0 / 16.0))
                pos = wv + plsc.cumsum(keep.astype(jnp.int32)) - 1
                plsc.store_scatter(cpk, [pos], v, mask=keep)
                plsc.store_scatter(cgate, [pos], g, mask=keep)
                wv = wv + plsc.all_reduce_population_count(keep)
            return wv
        wv = lax.fori_loop(0, _CR, _row, zi16)
        for kk in range(_B // 16):
            pad = wv + lane + kk * 16
            plsc.store_scatter(cpk, [pad], zi16)
            plsc.store_scatter(cgate, [pad], zf16)
        cnt = lax.reduce_max(wv, axes=(0,))
        nb = (cnt + (_B - 1)) // _B

        @pl.when(nb > 0)
        def _pro():
            build_idx(0, 0)
            pltpu.async_copy(m_hbm.at[sidx[0]], rows[0], gsem[0])

        def _iter(t, carry2):
            for p in (0, 1):
                b = t * 2 + p
                q = 1 - p
                @pl.when(b < nb)
                def _do():
                    bn = jnp.minimum(b + 1, nb - 1)
                    @pl.when(b >= 1)
                    def _drain_s():
                        pltpu.make_async_copy(rows[q], agg_sh.at[didx[q]],
                                              ssem[q]).wait()
                    build_idx(bn, q)
                    pltpu.async_copy(m_hbm.at[sidx[q]], rows[q], gsem[q])
                    pltpu.make_async_copy(m_hbm.at[sidx[p]], rows[p],
                                          gsem[p]).wait()
                    scale(b, p)
                    pltpu.async_copy(rows[p], agg_sh.at[didx[p]], ssem[p],
                                     add=True)
            return carry2
        lax.fori_loop(0, (_CH // _B + 2) // 2, _iter, 0)

        @pl.when(nb > 0)
        def _epi():
            @pl.when(nb % 2 == 0)
            def _e0():
                pltpu.make_async_copy(m_hbm.at[sidx[0]], rows[0], gsem[0]).wait()
                pltpu.make_async_copy(rows[1], agg_sh.at[didx[1]], ssem[1]).wait()
            @pl.when(nb % 2 == 1)
            def _e1():
                pltpu.make_async_copy(m_hbm.at[sidx[1]], rows[1], gsem[1]).wait()
                pltpu.make_async_copy(rows[0], agg_sh.at[didx[0]], ssem[0]).wait()
        return carry
    lax.fori_loop(0, _NCH, _chunk, 0)

    plsc.subcore_barrier()
    out_row = cid * _NP + row0
    pltpu.sync_copy(agg_sh.at[pl.ds(row0, _RPS)], out_hbm.at[pl.ds(out_row, _RPS)])


def _edge_call(pk, w1, w2, cell9, m):
    mesh = plsc.VectorSubcoreMesh(core_axis_name="c", subcore_axis_name="s")
    k = functools.partial(
        pl.kernel,
        out_type=jax.ShapeDtypeStruct((2 * _NP, _D), jnp.float32),
        mesh=mesh,
        compiler_params=pltpu.CompilerParams(needs_layout_passes=False,
                                             use_tc_tiling_on_sc=False),
        scratch_types=[
            pltpu.VMEM((_N,), jnp.int32),
            pltpu.VMEM((_N,), jnp.int32),
            pltpu.VMEM((_G * 9,), jnp.float32),
            pltpu.VMEM((_CR, _B), jnp.int32),
            pltpu.VMEM((_CH + _B,), jnp.int32),
            pltpu.VMEM((_CH + _B,), jnp.float32),
            pltpu.VMEM((_B,), jnp.int32),
            pltpu.VMEM((_B,), jnp.int32),
            pltpu.VMEM((_B,), jnp.int32),
            pltpu.VMEM((_B,), jnp.int32),
            pltpu.VMEM((_B, _D), jnp.float32),
            pltpu.VMEM((_B, _D), jnp.float32),
            pltpu.VMEM_SHARED((_NP, _D), jnp.float32),
            pltpu.SemaphoreType.DMA,
            pltpu.SemaphoreType.DMA,
            pltpu.SemaphoreType.DMA,
            pltpu.SemaphoreType.DMA,
        ],
    )(_edge_body)
    return k(pk, w1, w2, cell9, m)


def _head_body(h_ref, a0_ref, a1_ref, fx_ref, ma_ref, n2g_ref,
               w1_ref, b1_ref, w2_ref, b2_ref, y_ref,
               loss_ref, pred_ref, psum, csum):
    i = pl.program_id(0)

    @pl.when(i == 0)
    def _init():
        psum[...] = jnp.zeros_like(psum)
        csum[...] = jnp.zeros_like(csum)

    scale = (jnp.where(fx_ref[...] > 0, 0.0, 1.0)
             * jnp.where(ma_ref[...] > 0, 1.5, 1.0))
    h2 = (h_ref[...] + a0_ref[...] + a1_ref[...]) * scale
    t = _ssilu(jnp.dot(h2, w1_ref[...], preferred_element_type=jnp.float32)
               + b1_ref[...])
    p = jax.nn.sigmoid(jnp.dot(t, w2_ref[...], preferred_element_type=jnp.float32)
                       + b2_ref[...])
    ohg = (n2g_ref[...] == lax.broadcasted_iota(jnp.int32, (_R, _G), 1)
           ).astype(jnp.float32)
    dn = (((0,), (0,)), ((), ()))
    psum[...] += lax.dot_general(ohg, p, dn, preferred_element_type=jnp.float32)
    csum[...] += lax.dot_general(ohg, jnp.ones((_R, 1), jnp.float32), dn,
                                 preferred_element_type=jnp.float32)

    @pl.when(i == (_N // _R) - 1)
    def _fin():
        pred = psum[...] / jnp.maximum(csum[...], 1.0)
        pred_ref[...] = pred
        pc = jnp.clip(pred, 1e-7, 1.0 - 1e-7)
        y = y_ref[...]
        ll = y * jnp.log(pc) + (1.0 - y) * jnp.log(1.0 - pc)
        loss_ref[...] = jnp.full((1, 1), -jnp.mean(ll), jnp.float32)


def _head_call(h, a0, a1, fx, ma, n2g, w1, b1, w2, b2, y):
    grid = _N // _R
    return pl.pallas_call(
        _head_body,
        grid=(grid,),
        in_specs=[
            pl.BlockSpec((_R, _D), lambda i: (i, 0)),
            pl.BlockSpec((_R, _D), lambda i: (i, 0)),
            pl.BlockSpec((_R, _D), lambda i: (i, 0)),
            pl.BlockSpec((_R, 1), lambda i: (i, 0)),
            pl.BlockSpec((_R, 1), lambda i: (i, 0)),
            pl.BlockSpec((_R, 1), lambda i: (i, 0)),
            pl.BlockSpec((_D, _D // 2), lambda i: (0, 0)),
            pl.BlockSpec((1, _D // 2), lambda i: (0, 0)),
            pl.BlockSpec((_D // 2, 1), lambda i: (0, 0)),
            pl.BlockSpec((1, 1), lambda i: (0, 0)),
            pl.BlockSpec((_G, 1), lambda i: (0, 0)),
        ],
        out_specs=[
            pl.BlockSpec((1, 1), lambda i: (0, 0)),
            pl.BlockSpec((_G, 1), lambda i: (0, 0)),
        ],
        out_shape=[
            jax.ShapeDtypeStruct((1, 1), jnp.float32),
            jax.ShapeDtypeStruct((_G, 1), jnp.float32),
        ],
        scratch_shapes=[
            pltpu.VMEM((_G, 1), jnp.float32),
            pltpu.VMEM((_G, 1), jnp.float32),
        ],
    )(h, a0, a1, fx, ma, n2g, w1, b1, w2, b2, y)


def kernel(pos, cell, emb_table, W_edge, b_edge, W1, b1, W2, b2,
           atomic_numbers, node2graph, fixed, mask_ads, label, edge_index):
    an = atomic_numbers.astype(jnp.int32).reshape(_N, 1)
    n2g = node2graph.astype(jnp.int32)
    cell9 = cell.astype(jnp.float32).reshape(_G, 9)
    e2 = edge_index.astype(jnp.int32).reshape(2 * _EB, _B)
    h, m, w1t, w2t, pk = _node_call(an, n2g.reshape(_N, 1),
                                    pos.astype(jnp.float32), emb_table, W_edge,
                                    b_edge.reshape(1, _D), cell9, e2)
    agg2 = _edge_call(pk, w1t.reshape(_N), w2t.reshape(_N),
                      cell9.reshape(_G * 9), m)
    loss, pred = _head_call(
        h, agg2[:_N], agg2[_NP:_NP + _N],
        fixed.astype(jnp.int32).reshape(_N, 1),
        mask_ads.astype(jnp.int32).reshape(_N, 1),
        n2g.reshape(_N, 1),
        W1, b1.reshape(1, _D // 2), W2, b2.reshape(1, 1),
        label.astype(jnp.float32).reshape(_G, 1))
    return (loss.reshape(()), pred)


# speedup vs baseline: 1.3952x; 1.3952x over previous
import functools

import jax
import jax.numpy as jnp
from jax import lax
from jax.experimental import pallas as pl
from jax.experimental.pallas import tpu as pltpu
from jax.experimental.pallas import tpu_sc as plsc

_N, _G, _E, _D = 10000, 32, 320000, 128
_DH = _D // 2
_NP = 12800
_R = 400
_B = 80
_NBT = _E // 16 // _B
_RPS = _NP // 16
_EB = _E // _B
_CH = 2000


def _ssilu(x):
    return (x * jax.nn.sigmoid(x)) * (1.0 / 0.6)


def _node_body(an_ref, n2g_ref, pos_ref, emb_ref, we_ref, be_ref, cell_ref,
               es_ref, ed_ref,
               h_ref, mab_ref, wa_ref, wb_ref, pk_ref):
    c9 = cell_ref[...]
    a, b, c, d, e, f, g, h, i = [c9[:, k:k + 1] for k in range(9)]
    c11 = e * i - f * h
    c12 = -(d * i - f * g)
    c13 = d * h - e * g
    det = a * c11 + b * c12 + c * c13
    inv9 = jnp.concatenate([
        c11, -(b * i - c * h), b * f - c * e,
        c12, a * i - c * g, -(a * f - c * d),
        c13, -(a * h - b * g), a * e - b * d,
    ], axis=1) / det

    an = an_ref[...]
    oh = (an == lax.broadcasted_iota(jnp.int32, (_R, 100), 1)).astype(jnp.float32)
    hh = jnp.dot(oh, emb_ref[...], preferred_element_type=jnp.float32)
    h_ref[...] = hh
    m = _ssilu(jnp.dot(hh, we_ref[...], preferred_element_type=jnp.float32)
               + be_ref[...])
    mab_ref[0, :, :] = m[:, :_DH]
    mab_ref[1, :, :] = m[:, _DH:]

    n2g = n2g_ref[...]
    ohg = (n2g == lax.broadcasted_iota(jnp.int32, (_R, _G), 1)).astype(jnp.float32)
    ci = jnp.dot(ohg, inv9, preferred_element_type=jnp.float32)
    pos = pos_ref[...]
    q = []
    for j in range(3):
        v = (pos[:, 0:1] * ci[:, j:j + 1]
             + pos[:, 1:2] * ci[:, 3 + j:4 + j]
             + pos[:, 2:3] * ci[:, 6 + j:7 + j])
        v = v - jnp.floor(v)
        q.append(jnp.minimum(jnp.floor(v * 65536.0), 65535.0).astype(jnp.int32))
    wa_ref[...] = q[0] + q[1] * 65536
    wb_ref[...] = q[2] + n2g * 65536

    pk_ref[...] = ed_ref[...] * 16384 + es_ref[...]


def _node_call(an, n2g, pos, emb, we, be, cell9, e2):
    grid = _N // _R
    return pl.pallas_call(
        _node_body,
        grid=(grid,),
        in_specs=[
            pl.BlockSpec((_R, 1), lambda i: (i, 0)),
            pl.BlockSpec((_R, 1), lambda i: (i, 0)),
            pl.BlockSpec((_R, 3), lambda i: (i, 0)),
            pl.BlockSpec((100, _D), lambda i: (0, 0)),
            pl.BlockSpec((_D, _D), lambda i: (0, 0)),
            pl.BlockSpec((1, _D), lambda i: (0, 0)),
            pl.BlockSpec((_G, 9), lambda i: (0, 0)),
            pl.BlockSpec((_EB // (_N // _R), _B), lambda i: (i, 0)),
            pl.BlockSpec((_EB // (_N // _R), _B), lambda i: (i + _N // _R, 0)),
        ],
        out_specs=[
            pl.BlockSpec((_R, _D), lambda i: (i, 0)),
            pl.BlockSpec((2, _R, _DH), lambda i: (0, i, 0)),
            pl.BlockSpec((_R, 1), lambda i: (i, 0)),
            pl.BlockSpec((_R, 1), lambda i: (i, 0)),
            pl.BlockSpec((_EB // (_N // _R), _B), lambda i: (i, 0)),
        ],
        out_shape=[
            jax.ShapeDtypeStruct((_N, _D), jnp.float32),
            jax.ShapeDtypeStruct((2, _N, _DH), jnp.float32),
            jax.ShapeDtypeStruct((_N, 1), jnp.int32),
            jax.ShapeDtypeStruct((_N, 1), jnp.int32),
            jax.ShapeDtypeStruct((_EB, _B), jnp.int32),
        ],
    )(an, n2g, pos, emb, we, be, cell9, e2, e2)


def _splat_i32(v):
    return jnp.full((16,), v, dtype=jnp.int32)


def _edge_body(pk_hbm, wa_hbm, wb_hbm, cell_hbm, m2_hbm, out_hbm,
               wa_v, wb_v, cell_v, pk_c, cpk, cgate, sidx0, sidx1,
               didx0, didx1, rows0, rows1, agg_sh,
               gsem0, gsem1, ssem0, ssem1):
    cid = lax.axis_index("c")
    sid = lax.axis_index("s")

    pltpu.sync_copy(wa_hbm, wa_v)
    pltpu.sync_copy(wb_hbm, wb_v)
    pltpu.sync_copy(cell_hbm, cell_v)
    pltpu.sync_copy(pk_hbm.at[pl.ds(sid * _NBT, _NBT)], pk_c)

    def _zf(r, carry):
        for q in range(_DH // 16):
            rows0[r, pl.ds(q * 16, 16)] = jnp.zeros((16,), jnp.float32)
        return carry
    lax.fori_loop(0, _B, _zf, 0)
    row0 = sid * _RPS

    def _zc(t, carry):
        pltpu.sync_copy(rows0, agg_sh.at[pl.ds(row0 + t * _B, _B)])
        return carry
    lax.fori_loop(0, _RPS // _B, _zc, 0)
    plsc.subcore_barrier()

    moff = cid * _N
    sidx = (sidx0, sidx1)
    didx = (didx0, didx1)
    rows = (rows0, rows1)
    gsem = (gsem0, gsem1)
    ssem = (ssem0, ssem1)
    zi16 = jnp.zeros((16,), jnp.int32)
    zf16 = jnp.zeros((16,), jnp.float32)
    lane = lax.iota(jnp.int32, 16)
    u16scale = jnp.full((16,), 1.0 / 65536.0, jnp.float32)

    def build_idx(b, p):
        off = pl.multiple_of(b * _B, 16)
        def _bi(k, carry):
            v = cpk[pl.ds(off + k * 16, 16)]
            sidx[p][pl.ds(k * 16, 16)] = (v & 16383) + moff
            didx[p][pl.ds(k * 16, 16)] = lax.shift_right_logical(v, 14)
            return carry
        lax.fori_loop(0, _B // 16, _bi, 0)

    def scale(b, p):
        boff = b * _B
        def _s(e4, carry):
            for u in range(4):
                ei = e4 * 4 + u
                gs = plsc.load_gather(cgate, [_splat_i32(boff + ei)])
                for q in range(_DH // 16):
                    rows[p][ei, pl.ds(q * 16, 16)] = rows[p][ei, pl.ds(q * 16, 16)] * gs
            return carry
        lax.fori_loop(0, _B // 4, _s, 0)

    def _chunk(c, carry):
        crow = c * (_CH // _B)
        def _row(r, wv):
            for k in range(_B // 16):
                v = pk_c[crow + r, pl.ds(k * 16, 16)]
                s16 = v & 16383
                d16 = lax.shift_right_logical(v, 14)
                wsa = plsc.load_gather(wa_v, [s16])
                wda = plsc.load_gather(wa_v, [d16])
                wsb = plsc.load_gather(wb_v, [s16])
                wdb = plsc.load_gather(wb_v, [d16])
                df = []
                for (ws, wd, hi) in ((wsa, wda, 0), (wsa, wda, 1), (wsb, wdb, 0)):
                    if hi == 0:
                        qs = (ws & 65535).astype(jnp.float32)
                        qd = (wd & 65535).astype(jnp.float32)
                    else:
                        qs = lax.shift_right_logical(ws, 16).astype(jnp.float32)
                        qd = lax.shift_right_logical(wd, 16).astype(jnp.float32)
                    x = (qd - qs) * u16scale
                    x = jnp.where(x > 0.5, x - 1.0,
                                  jnp.where(x < -0.5, x + 1.0, x))
                    df.append(x)
                g9 = lax.shift_right_logical(wsb, 16) * 9
                s2 = jnp.full((16,), 1e-12, jnp.float32)
                for j3 in range(3):
                    dv = jnp.zeros((16,), jnp.float32)
                    for comp in range(3):
                        ce = plsc.load_gather(cell_v, [g9 + _splat_i32(comp * 3 + j3)])
                        dv = dv + df[comp] * ce
                    s2 = s2 + dv * dv
                keep = s2 < 16.0
                g = jnp.exp(s2 * (-1.0 / 16.0))
                pos = wv + plsc.cumsum(keep.astype(jnp.int32)) - 1
                plsc.store_scatter(cpk, [pos], v, mask=keep)
                plsc.store_scatter(cgate, [pos], g, mask=keep)
                wv = wv + plsc.all_reduce_population_count(keep)
            return wv
        wv = lax.fori_loop(0, _CH // _B, _row, zi16)
        for kk in range(_B // 16):
            pad = wv + lane + kk * 16
            plsc.store_scatter(cpk, [pad], zi16)
            plsc.store_scatter(cgate, [pad], zf16)
        cnt = lax.reduce_max(wv, axes=(0,))
        nb = (cnt + (_B - 1)) // _B

        @pl.when(nb > 0)
        def _pro():
            build_idx(0, 0)
            pltpu.async_copy(m2_hbm.at[sidx[0]], rows[0], gsem[0])

        def _iter(t, carry2):
            for p in (0, 1):
                b = t * 2 + p
                q = 1 - p
                @pl.when(b < nb)
                def _do():
                    bn = jnp.minimum(b + 1, nb - 1)
                    @pl.when(b >= 1)
                    def _drain_s():
                        pltpu.make_async_copy(rows[q], agg_sh.at[didx[q]],
                                              ssem[q]).wait()
                    build_idx(bn, q)
                    pltpu.async_copy(m2_hbm.at[sidx[q]], rows[q], gsem[q])
                    pltpu.make_async_copy(m2_hbm.at[sidx[p]], rows[p],
                                          gsem[p]).wait()
                    scale(b, p)
                    pltpu.async_copy(rows[p], agg_sh.at[didx[p]], ssem[p],
                                     add=True)
            return carry2
        lax.fori_loop(0, (_CH // _B + 2) // 2, _iter, 0)

        @pl.when(nb > 0)
        def _epi():
            @pl.when(nb % 2 == 0)
            def _e0():
                pltpu.make_async_copy(m2_hbm.at[sidx[0]], rows[0], gsem[0]).wait()
                pltpu.make_async_copy(rows[1], agg_sh.at[didx[1]], ssem[1]).wait()
            @pl.when(nb % 2 == 1)
            def _e1():
                pltpu.make_async_copy(m2_hbm.at[sidx[1]], rows[1], gsem[1]).wait()
                pltpu.make_async_copy(rows[0], agg_sh.at[didx[0]], ssem[0]).wait()
        return carry
    lax.fori_loop(0, _NBT // (_CH // _B), _chunk, 0)

    plsc.subcore_barrier()
    out_row = cid * _NP + row0
    pltpu.sync_copy(agg_sh.at[pl.ds(row0, _RPS)], out_hbm.at[pl.ds(out_row, _RPS)])


def _edge_call(pk, wa, wb, cell9, m2):
    mesh = plsc.VectorSubcoreMesh(core_axis_name="c", subcore_axis_name="s")
    k = functools.partial(
        pl.kernel,
        out_type=jax.ShapeDtypeStruct((2 * _NP, _DH), jnp.float32),
        mesh=mesh,
        compiler_params=pltpu.CompilerParams(needs_layout_passes=False,
                                             use_tc_tiling_on_sc=False),
        scratch_types=[
            pltpu.VMEM((_N,), jnp.int32),
            pltpu.VMEM((_N,), jnp.int32),
            pltpu.VMEM((_G * 9,), jnp.float32),
            pltpu.VMEM((_NBT, _B), jnp.int32),
            pltpu.VMEM((_CH + _B,), jnp.int32),
            pltpu.VMEM((_CH + _B,), jnp.float32),
            pltpu.VMEM((_B,), jnp.int32),
            pltpu.VMEM((_B,), jnp.int32),
            pltpu.VMEM((_B,), jnp.int32),
            pltpu.VMEM((_B,), jnp.int32),
            pltpu.VMEM((_B, _DH), jnp.float32),
            pltpu.VMEM((_B, _DH), jnp.float32),
            pltpu.VMEM_SHARED((_NP, _DH), jnp.float32),
            pltpu.SemaphoreType.DMA,
            pltpu.SemaphoreType.DMA,
            pltpu.SemaphoreType.DMA,
            pltpu.SemaphoreType.DMA,
        ],
    )(_edge_body)
    return k(pk, wa, wb, cell9, m2)


def _head_body(h_ref, a0_ref, a1_ref, fx_ref, ma_ref, n2g_ref,
               w1_ref, b1_ref, w2_ref, b2_ref, y_ref,
               loss_ref, pred_ref, psum, csum):
    i = pl.program_id(0)

    @pl.when(i == 0)
    def _init():
        psum[...] = jnp.zeros_like(psum)
        csum[...] = jnp.zeros_like(csum)

    scale = (jnp.where(fx_ref[...] > 0, 0.0, 1.0)
             * jnp.where(ma_ref[...] > 0, 1.5, 1.0))
    agg = jnp.concatenate([a0_ref[...], a1_ref[...]], axis=1)
    h2 = (h_ref[...] + agg) * scale
    t = _ssilu(jnp.dot(h2, w1_ref[...], preferred_element_type=jnp.float32)
               + b1_ref[...])
    p = jax.nn.sigmoid(jnp.dot(t, w2_ref[...], preferred_element_type=jnp.float32)
                       + b2_ref[...])
    ohg = (n2g_ref[...] == lax.broadcasted_iota(jnp.int32, (_R, _G), 1)
           ).astype(jnp.float32)
    dn = (((0,), (0,)), ((), ()))
    psum[...] += lax.dot_general(ohg, p, dn, preferred_element_type=jnp.float32)
    csum[...] += lax.dot_general(ohg, jnp.ones((_R, 1), jnp.float32), dn,
                                 preferred_element_type=jnp.float32)

    @pl.when(i == (_N // _R) - 1)
    def _fin():
        pred = psum[...] / jnp.maximum(csum[...], 1.0)
        pred_ref[...] = pred
        pc = jnp.clip(pred, 1e-7, 1.0 - 1e-7)
        y = y_ref[...]
        ll = y * jnp.log(pc) + (1.0 - y) * jnp.log(1.0 - pc)
        loss_ref[...] = jnp.full((1, 1), -jnp.mean(ll), jnp.float32)


def _head_call(h, a0, a1, fx, ma, n2g, w1, b1, w2, b2, y):
    grid = _N // _R
    return pl.pallas_call(
        _head_body,
        grid=(grid,),
        in_specs=[
            pl.BlockSpec((_R, _D), lambda i: (i, 0)),
            pl.BlockSpec((_R, _DH), lambda i: (i, 0)),
            pl.BlockSpec((_R, _DH), lambda i: (i + _NP // _R, 0)),
            pl.BlockSpec((_R, 1), lambda i: (i, 0)),
            pl.BlockSpec((_R, 1), lambda i: (i, 0)),
            pl.BlockSpec((_R, 1), lambda i: (i, 0)),
            pl.BlockSpec((_D, _D // 2), lambda i: (0, 0)),
            pl.BlockSpec((1, _D // 2), lambda i: (0, 0)),
            pl.BlockSpec((_D // 2, 1), lambda i: (0, 0)),
            pl.BlockSpec((1, 1), lambda i: (0, 0)),
            pl.BlockSpec((_G, 1), lambda i: (0, 0)),
        ],
        out_specs=[
            pl.BlockSpec((1, 1), lambda i: (0, 0)),
            pl.BlockSpec((_G, 1), lambda i: (0, 0)),
        ],
        out_shape=[
            jax.ShapeDtypeStruct((1, 1), jnp.float32),
            jax.ShapeDtypeStruct((_G, 1), jnp.float32),
        ],
        scratch_shapes=[
            pltpu.VMEM((_G, 1), jnp.float32),
            pltpu.VMEM((_G, 1), jnp.float32),
        ],
    )(h, a0, a1, fx, ma, n2g, w1, b1, w2, b2, y)


def kernel(pos, cell, emb_table, W_edge, b_edge, W1, b1, W2, b2,
           atomic_numbers, node2graph, fixed, mask_ads, label, edge_index):
    an = atomic_numbers.astype(jnp.int32).reshape(_N, 1)
    n2g = node2graph.astype(jnp.int32)
    cell9 = cell.astype(jnp.float32).reshape(_G, 9)
    e2 = edge_index.astype(jnp.int32).reshape(2 * _EB, _B)
    h, mAB, wa, wb, pk = _node_call(an, n2g.reshape(_N, 1),
                                    pos.astype(jnp.float32), emb_table, W_edge,
                                    b_edge.reshape(1, _D), cell9, e2)
    m2 = mAB.reshape(2 * _N, _DH)
    agg2 = _edge_call(pk, wa.reshape(_N), wb.reshape(_N),
                      cell9.reshape(_G * 9), m2)
    loss, pred = _head_call(
        h, agg2, agg2,
        fixed.astype(jnp.int32).reshape(_N, 1),
        mask_ads.astype(jnp.int32).reshape(_N, 1),
        n2g.reshape(_N, 1),
        W1, b1.reshape(1, _D // 2), W2, b2.reshape(1, 1),
        label.astype(jnp.float32).reshape(_G, 1))
    return (loss.reshape(()), pred)
